# baseline (speedup 1.0000x reference)
import functools

import jax
import jax.numpy as jnp
from jax import lax
from jax.experimental import pallas as pl
from jax.experimental.pallas import tpu as pltpu
from jax.experimental.pallas import tpu_sc as plsc

_VEC = 16
_CS = 8
_NBUF = 3


def kernel(x, pe):
    B, S, D = x.shape
    info = plsc.get_sparse_core_info()
    nw = info.num_cores * info.num_subcores
    s_per_w = S // nw
    n_chunks = s_per_w // _CS
    cw = _CS * D

    xf = x.reshape(B, S * D)
    pef = pe.reshape(S * D)
    mesh = plsc.VectorSubcoreMesh(core_axis_name="c", subcore_axis_name="s")

    @functools.partial(
        pl.kernel,
        mesh=mesh,
        out_type=jax.ShapeDtypeStruct((B, S * D), jnp.float32),
        scratch_types=(
            [pltpu.VMEM((cw,), jnp.float32) for _ in range(_NBUF)]
            + [pltpu.VMEM((B, cw), jnp.float32) for _ in range(_NBUF)]
            + [pltpu.SemaphoreType.DMA for _ in range(2 * _NBUF)]
        ),
    )
    def sc_add(x_hbm, pe_hbm, out_hbm, *scratch):
        pe_bufs = scratch[:_NBUF]
        x_bufs = scratch[_NBUF : 2 * _NBUF]
        in_sems = scratch[2 * _NBUF : 3 * _NBUF]
        out_sems = scratch[3 * _NBUF : 4 * _NBUF]
        wid = lax.axis_index("s") * info.num_cores + lax.axis_index("c")
        s_base = wid * s_per_w

        def start_in(c, buf):
            off = (s_base + c * _CS) * D
            cps = [pltpu.async_copy(pe_hbm.at[pl.ds(off, cw)], pe_bufs[buf], in_sems[buf])]
            for b in range(B):
                cps.append(
                    pltpu.async_copy(x_hbm.at[b, pl.ds(off, cw)], x_bufs[buf].at[b], in_sems[buf])
                )
            return cps

        def start_out(c, buf):
            off = (s_base + c * _CS) * D
            return [
                pltpu.async_copy(x_bufs[buf].at[b], out_hbm.at[b, pl.ds(off, cw)], out_sems[buf])
                for b in range(B)
            ]

        handles_in = {0: start_in(0, 0)}
        handles_out = {}
        for c in range(n_chunks):
            buf = c % _NBUF
            for h in handles_in.pop(c):
                h.wait()
            if c + 1 < n_chunks:
                if c - (_NBUF - 1) in handles_out:
                    for h in handles_out.pop(c - (_NBUF - 1)):
                        h.wait()
                handles_in[c + 1] = start_in(c + 1, (c + 1) % _NBUF)

            pe_b = pe_bufs[buf]
            x_b = x_bufs[buf]

            def vec_body(j, _):
                s0 = pl.ds(j * 2 * _VEC, _VEC)
                s1 = pl.ds(j * 2 * _VEC + _VEC, _VEC)
                p0 = pe_b[s0]
                p1 = pe_b[s1]
                for b in range(B):
                    x_b[b, s0] += p0
                    x_b[b, s1] += p1
                return 0

            lax.fori_loop(0, cw // (2 * _VEC), vec_body, 0)
            handles_out[c] = start_out(c, buf)
        for c in sorted(handles_out):
            for h in handles_out[c]:
                h.wait()

    out = sc_add(xf, pef)
    return out.reshape(B, S, D)


# speedup vs baseline: 1.2774x; 1.2774x over previous
import functools

import jax
import jax.numpy as jnp
from jax import lax
from jax.experimental import pallas as pl
from jax.experimental.pallas import tpu as pltpu
from jax.experimental.pallas import tpu_sc as plsc

_VEC = 16
_CS = 8
_NBUF = 3


def kernel(x, pe):
    B, S, D = x.shape
    info = plsc.get_sparse_core_info()
    nw = info.num_cores * info.num_subcores
    s_per_w = S // nw
    n_chunks = s_per_w // _CS
    cw = _CS * D

    xf = x.reshape(B, S * D)
    pef = pe.reshape(S * D)
    mesh = plsc.VectorSubcoreMesh(core_axis_name="c", subcore_axis_name="s")

    @functools.partial(
        pl.kernel,
        mesh=mesh,
        out_type=jax.ShapeDtypeStruct((B, S * D), jnp.float32),
        scratch_types=(
            [pltpu.VMEM((cw,), jnp.float32) for _ in range(_NBUF)]
            + [pltpu.VMEM((B, cw), jnp.float32) for _ in range(_NBUF)]
            + [pltpu.SemaphoreType.DMA for _ in range(2 * _NBUF)]
        ),
    )
    def sc_add(x_hbm, pe_hbm, out_hbm, *scratch):
        pe_bufs = scratch[:_NBUF]
        x_bufs = scratch[_NBUF : 2 * _NBUF]
        in_sems = scratch[2 * _NBUF : 3 * _NBUF]
        out_sems = scratch[3 * _NBUF : 4 * _NBUF]
        wid = lax.axis_index("s") * info.num_cores + lax.axis_index("c")
        s_base = wid * s_per_w

        def start_in(c, buf):
            off = (s_base + c * _CS) * D
            cps = [pltpu.async_copy(pe_hbm.at[pl.ds(off, cw)], pe_bufs[buf], in_sems[buf])]
            for b in range(B):
                cps.append(
                    pltpu.async_copy(x_hbm.at[b, pl.ds(off, cw)], x_bufs[buf].at[b], in_sems[buf])
                )
            return cps

        def start_out(c, buf):
            off = (s_base + c * _CS) * D
            return [
                pltpu.async_copy(x_bufs[buf].at[b], out_hbm.at[b, pl.ds(off, cw)], out_sems[buf])
                for b in range(B)
            ]

        handles_in = {0: start_in(0, 0)}
        handles_out = {}
        for c in range(n_chunks):
            buf = c % _NBUF
            for h in handles_in.pop(c):
                h.wait()
            if c + 1 < n_chunks:
                if c - (_NBUF - 1) in handles_out:
                    for h in handles_out.pop(c - (_NBUF - 1)):
                        h.wait()
                handles_in[c + 1] = start_in(c + 1, (c + 1) % _NBUF)

            pe_b = pe_bufs[buf]
            x_b = x_bufs[buf]

            @plsc.parallel_loop(0, cw, step=_VEC, unroll=8)
            def _(i):
                sl = pl.ds(i, _VEC)
                pv = pe_b[sl]
                for b in range(B):
                    x_b[b, sl] += pv
            handles_out[c] = start_out(c, buf)
        for c in sorted(handles_out):
            for h in handles_out[c]:
                h.wait()

    out = sc_add(xf, pef)
    return out.reshape(B, S, D)


# speedup vs baseline: 3.4746x; 2.7201x over previous
import functools

import jax
import jax.numpy as jnp
from jax import lax
from jax.experimental import pallas as pl
from jax.experimental.pallas import tpu as pltpu
from jax.experimental.pallas import tpu_sc as plsc

_VEC = 16
_CS = 8
_NBUF = 3


def kernel(x, pe):
    B, S, D = x.shape
    info = plsc.get_sparse_core_info()
    nw = info.num_cores * info.num_subcores
    s_per_w = S // nw
    n_chunks = s_per_w // _CS
    assert D & (D - 1) == 0
    d_shift = D.bit_length() - 1
    mesh = plsc.VectorSubcoreMesh(core_axis_name="c", subcore_axis_name="s")

    @functools.partial(
        pl.kernel,
        mesh=mesh,
        out_type=jax.ShapeDtypeStruct((B, S, D), jnp.float32),
        scratch_types=(
            [pltpu.VMEM((_CS, D), jnp.float32) for _ in range(_NBUF)]
            + [pltpu.VMEM((B, _CS, D), jnp.float32) for _ in range(_NBUF)]
            + [pltpu.SemaphoreType.DMA for _ in range(2 * _NBUF)]
        ),
    )
    def sc_add(x_hbm, pe_hbm, out_hbm, *scratch):
        pe_bufs = scratch[:_NBUF]
        x_bufs = scratch[_NBUF : 2 * _NBUF]
        in_sems = scratch[2 * _NBUF : 3 * _NBUF]
        out_sems = scratch[3 * _NBUF : 4 * _NBUF]
        wid = lax.axis_index("s") * info.num_cores + lax.axis_index("c")
        s_base = wid * s_per_w

        def start_in(c, buf):
            s0 = s_base + c * _CS
            cps = [pltpu.async_copy(pe_hbm.at[pl.ds(s0, _CS), :], pe_bufs[buf], in_sems[buf])]
            for b in range(B):
                cps.append(
                    pltpu.async_copy(
                        x_hbm.at[b, pl.ds(s0, _CS), :], x_bufs[buf].at[b], in_sems[buf]
                    )
                )
            return cps

        def start_out(c, buf):
            s0 = s_base + c * _CS
            return [
                pltpu.async_copy(
                    x_bufs[buf].at[b], out_hbm.at[b, pl.ds(s0, _CS), :], out_sems[buf]
                )
                for b in range(B)
            ]

        handles_in = {0: start_in(0, 0)}
        handles_out = {}
        for c in range(n_chunks):
            buf = c % _NBUF
            for h in handles_in.pop(c):
                h.wait()
            if c + 1 < n_chunks:
                if c - (_NBUF - 1) in handles_out:
                    for h in handles_out.pop(c - (_NBUF - 1)):
                        h.wait()
                handles_in[c + 1] = start_in(c + 1, (c + 1) % _NBUF)

            pe_b = pe_bufs[buf]
            x_b = x_bufs[buf]

            @plsc.parallel_loop(0, _CS * D, step=_VEC, unroll=8)
            def _(i):
                r = lax.shift_right_logical(i, d_shift)
                col = pl.multiple_of(lax.bitwise_and(i, D - 1), _VEC)
                sl = pl.ds(col, _VEC)
                pv = pe_b[r, sl]
                for b in range(B):
                    x_b[b, r, sl] += pv

            handles_out[c] = start_out(c, buf)
        for c in sorted(handles_out):
            for h in handles_out[c]:
                h.wait()

    return sc_add(x, pe)


# speedup vs baseline: 3.5288x; 1.0156x over previous
import functools

import jax
import jax.numpy as jnp
from jax import lax
from jax.experimental import pallas as pl
from jax.experimental.pallas import tpu as pltpu
from jax.experimental.pallas import tpu_sc as plsc

_VEC = 16
_CS = 8
_NBUF = 3


def kernel(x, pe):
    B, S, D = x.shape
    info = plsc.get_sparse_core_info()
    nw = info.num_cores * info.num_subcores
    s_per_w = S // nw
    n_chunks = s_per_w // _CS
    assert D & (D - 1) == 0
    d_shift = D.bit_length() - 1
    mesh = plsc.VectorSubcoreMesh(core_axis_name="c", subcore_axis_name="s")

    @functools.partial(
        pl.kernel,
        mesh=mesh,
        out_type=jax.ShapeDtypeStruct((B, S, D), jnp.float32),
        scratch_types=(
            [pltpu.VMEM((_CS, D), jnp.float32) for _ in range(_NBUF)]
            + [pltpu.VMEM((B, _CS, D), jnp.float32) for _ in range(_NBUF)]
            + [pltpu.SemaphoreType.DMA for _ in range(2 * _NBUF)]
        ),
    )
    def sc_add(x_hbm, pe_hbm, out_hbm, *scratch):
        pe_bufs = scratch[:_NBUF]
        x_bufs = scratch[_NBUF : 2 * _NBUF]
        in_sems = scratch[2 * _NBUF : 3 * _NBUF]
        out_sems = scratch[3 * _NBUF : 4 * _NBUF]
        wid = lax.axis_index("s") * info.num_cores + lax.axis_index("c")
        s_base = wid * s_per_w

        def start_in(c, buf):
            s0 = s_base + c * _CS
            return [
                pltpu.async_copy(pe_hbm.at[pl.ds(s0, _CS), :], pe_bufs[buf], in_sems[buf]),
                pltpu.async_copy(x_hbm.at[:, pl.ds(s0, _CS), :], x_bufs[buf], in_sems[buf]),
            ]

        def start_out(c, buf):
            s0 = s_base + c * _CS
            return [
                pltpu.async_copy(x_bufs[buf], out_hbm.at[:, pl.ds(s0, _CS), :], out_sems[buf])
            ]

        handles_in = {0: start_in(0, 0)}
        handles_out = {}
        for c in range(n_chunks):
            buf = c % _NBUF
            for h in handles_in.pop(c):
                h.wait()
            if c + 1 < n_chunks:
                if c - (_NBUF - 1) in handles_out:
                    for h in handles_out.pop(c - (_NBUF - 1)):
                        h.wait()
                handles_in[c + 1] = start_in(c + 1, (c + 1) % _NBUF)

            pe_b = pe_bufs[buf]
            x_b = x_bufs[buf]

            @plsc.parallel_loop(0, _CS * D, step=_VEC, unroll=8)
            def _(i):
                r = lax.shift_right_logical(i, d_shift)
                col = pl.multiple_of(lax.bitwise_and(i, D - 1), _VEC)
                sl = pl.ds(col, _VEC)
                pv = pe_b[r, sl]
                for b in range(B):
                    x_b[b, r, sl] += pv

            handles_out[c] = start_out(c, buf)
        for c in sorted(handles_out):
            for h in handles_out[c]:
                h.wait()

    return sc_add(x, pe)
